# baseline (speedup 1.0000x reference)
import jax
import jax.numpy as jnp
from jax import lax
from jax.experimental import pallas as pl
from jax.experimental.pallas import tpu as pltpu
from jax.experimental.pallas import tpu_sc as plsc

B = 16384
V = 100000
D = 64
PD = 64
NNEG = 5

NC = 2
NS = 16
NW = NC * NS
EPT = B // NW
CH = 512
L = 16


OB = 8192
OFF = 7 * OB


def _repack_kernel(t1_ref, t2_ref, lo_ref, hi_ref, out_ref):
    t1 = lax.dot_general(t1_ref[...], lo_ref[...], (((0,), (0,)), ((), ())),
                         preferred_element_type=jnp.float32)
    t2 = lax.dot_general(t2_ref[...], hi_ref[...], (((0,), (0,)), ((), ())),
                         preferred_element_type=jnp.float32)
    lane = lax.broadcasted_iota(jnp.int32, (OB, 2 * D), 1)
    out_ref[...] = jnp.where(lane >= D, t2, t1)


@jax.jit
def _repack(tab_t, lo, hi):
    return pl.pallas_call(
        _repack_kernel,
        grid=(OFF // OB,),
        in_specs=[pl.BlockSpec((D, OB), lambda i: (0, i)),
                  pl.BlockSpec((D, OB),
                               lambda i: (0, jnp.minimum(i + OFF // OB,
                                                         (V - 1) // OB))),
                  pl.BlockSpec((D, 2 * D), lambda i: (0, 0)),
                  pl.BlockSpec((D, 2 * D), lambda i: (0, 0))],
        out_specs=pl.BlockSpec((OB, 2 * D), lambda i: (i, 0)),
        out_shape=jax.ShapeDtypeStruct((OFF, 2 * D), jnp.float32),
    )(tab_t, tab_t, lo, hi)


def _remap(idx):
    two = idx + idx
    return jnp.where(idx >= OFF, two - (2 * OFF - 1), two)


def _sc_gather_u_kernel(pu, u_tab, pair_out, pubuf, idxbuf,
                        rows_a, rows_b, gs0, gs1, ws0, ws1):
    wid = lax.axis_index("s") * NC + lax.axis_index("c")
    e0 = wid * EPT

    pltpu.sync_copy(pu.at[pl.ds(e0, EPT)], pubuf.at[pl.ds(0, EPT)])
    pltpu.sync_copy(pu.at[pl.ds(B + e0, EPT)], pubuf.at[pl.ds(EPT, EPT)])

    lane = lax.iota(jnp.int32, L)
    for j in range(2 * EPT // L):
        p = lane + j * L
        i = p >> 1
        slot = p & 1
        idxbuf[pl.ds(j * L, L)] = _remap(
            plsc.load_gather(pubuf, [i + slot * EPT]))

    base = wid * 2 * EPT
    bufs = (rows_a, rows_b)
    gsems = (gs0, gs1)
    wsems = (ws0, ws1)
    g0 = pltpu.async_copy(u_tab.at[idxbuf.at[pl.ds(0, CH)]], rows_a, gs0)
    g1 = pltpu.async_copy(u_tab.at[idxbuf.at[pl.ds(CH, CH)]], rows_b, gs1)
    g0.wait()
    w0 = pltpu.async_copy(rows_a, pair_out.at[pl.ds(base, CH)], ws0)
    g1.wait()
    w1 = pltpu.async_copy(rows_b, pair_out.at[pl.ds(base + CH, CH)], ws1)
    w0.wait()
    w1.wait()


@jax.jit
def _sc_gather_u(pu, u_tab):
    mesh = plsc.VectorSubcoreMesh(core_axis_name="c", subcore_axis_name="s")
    f = pl.kernel(
        _sc_gather_u_kernel,
        name="sc_gather_u",
        out_type=jax.ShapeDtypeStruct((2 * B, D), jnp.float32),
        mesh=mesh,
        scratch_types=[
            pltpu.VMEM((2 * EPT,), jnp.int32),
            pltpu.VMEM((2 * EPT,), jnp.int32),
            pltpu.VMEM((CH, D), jnp.float32),
            pltpu.VMEM((CH, D), jnp.float32),
            pltpu.SemaphoreType.DMA,
            pltpu.SemaphoreType.DMA,
            pltpu.SemaphoreType.DMA,
            pltpu.SemaphoreType.DMA,
        ],
        compiler_params=pltpu.CompilerParams(use_tc_tiling_on_sc=False,
                                             needs_layout_passes=False),
    )
    return f(pu, u_tab)


def _sc_gather_v_kernel(pv, ng, v_tab, vc0_out, vc1_out, vc2_out,
                        vbuf, idxbuf, rows_a, rows_b, gs0, gs1, ws0, ws1):
    wid = lax.axis_index("s") * NC + lax.axis_index("c")
    e0 = wid * EPT

    pltpu.sync_copy(pv.at[pl.ds(e0, EPT)], vbuf.at[pl.ds(0, EPT)])
    for j in range(NNEG):
        pltpu.sync_copy(ng.at[pl.ds(j * B + e0, EPT)],
                        vbuf.at[pl.ds((1 + j) * EPT, EPT)])

    lane = lax.iota(jnp.int32, L)
    for j in range(2 * EPT // L):
        p = lane + j * L
        i = p >> 1
        slot = p & 1
        a0 = i + slot * EPT
        a1 = i + (2 + slot) * EPT
        a2 = i + (4 + slot) * EPT
        idxbuf[pl.ds(j * L, L)] = _remap(plsc.load_gather(vbuf, [a0]))
        idxbuf[pl.ds(2 * EPT + j * L, L)] = _remap(
            plsc.load_gather(vbuf, [a1]))
        idxbuf[pl.ds(4 * EPT + j * L, L)] = _remap(
            plsc.load_gather(vbuf, [a2]))

    outs = (vc0_out, vc1_out, vc2_out)
    bufs = (rows_a, rows_b)
    gsems = (gs0, gs1)
    wsems = (ws0, ws1)
    NCHV = 6 * EPT // CH
    gds, wds = [None] * NCHV, [None] * NCHV
    for c in range(NCHV):
        b = c % 2
        if c >= 2:
            wds[c - 2].wait()
        gds[c] = pltpu.async_copy(
            v_tab.at[idxbuf.at[pl.ds(c * CH, CH)]], bufs[b], gsems[b])
        if c >= 1:
            gds[c - 1].wait()
            pc = c - 1
            seg, cc = pc // 2, pc % 2
            dst = wid * 2 * EPT + cc * CH
            wds[pc] = pltpu.async_copy(
                bufs[pc % 2], outs[seg].at[pl.ds(dst, CH)], wsems[pc % 2])
    gds[NCHV - 1].wait()
    seg, cc = (NCHV - 1) // 2, (NCHV - 1) % 2
    dst = wid * 2 * EPT + cc * CH
    wds[NCHV - 1] = pltpu.async_copy(
        bufs[(NCHV - 1) % 2], outs[seg].at[pl.ds(dst, CH)],
        wsems[(NCHV - 1) % 2])
    wds[NCHV - 2].wait()
    wds[NCHV - 1].wait()


@jax.jit
def _sc_gather_v(pv, ng, v_tab):
    mesh = plsc.VectorSubcoreMesh(core_axis_name="c", subcore_axis_name="s")
    f = pl.kernel(
        _sc_gather_v_kernel,
        out_type=[jax.ShapeDtypeStruct((2 * B, D), jnp.float32)
                  for _ in range(3)],
        mesh=mesh,
        scratch_types=[
            pltpu.VMEM((6 * EPT,), jnp.int32),
            pltpu.VMEM((6 * EPT,), jnp.int32),
            pltpu.VMEM((CH, D), jnp.float32),
            pltpu.VMEM((CH, D), jnp.float32),
            pltpu.SemaphoreType.DMA,
            pltpu.SemaphoreType.DMA,
            pltpu.SemaphoreType.DMA,
            pltpu.SemaphoreType.DMA,
        ],
        compiler_params=pltpu.CompilerParams(use_tc_tiling_on_sc=False,
                                             needs_layout_passes=False),
    )
    return f(pv, ng, v_tab)


def _log_sigmoid(x):
    return jnp.minimum(x, 0.0) - jnp.log1p(jnp.exp(-jnp.abs(x)))


BS = 8192


def _tc_loss_kernel(pair_ref, vc0_ref, vc1_ref, vc2_ref, Wcat_ref, bcat_ref,
                    H_ref, out_ref):
    i = pl.program_id(0)

    relcat = lax.dot_general(pair_ref[...], Wcat_ref[...],
                             (((1,), (1,)), ((), ())),
                             preferred_element_type=jnp.float32)
    relcat += bcat_ref[...]

    pcat = jnp.concatenate([vc0_ref[...] * relcat,
                            vc1_ref[...] * relcat,
                            vc2_ref[...] * relcat], axis=0)
    st = lax.dot_general(H_ref[...], pcat, (((0,), (1,)), ((), ())),
                         preferred_element_type=jnp.float32)

    row = lax.broadcasted_iota(jnp.int32, (2, 3 * BS), 0)
    colid = lax.broadcasted_iota(jnp.int32, (2, 3 * BS), 1)
    sgn = jnp.where((row == 0) & (colid < BS), 1.0, -1.0)
    tot = jnp.sum(_log_sigmoid(st * sgn))

    @pl.when(i == 0)
    def _():
        out_ref[...] = jnp.zeros((1, 1), jnp.float32)

    out_ref[...] += jnp.broadcast_to(tot, (1, 1))


@jax.jit
def _tc_loss(pair, vc0, vc1, vc2, Wcat, bcat, H):
    return pl.pallas_call(
        _tc_loss_kernel,
        grid=(B // BS,),
        in_specs=[
            pl.BlockSpec((BS, 2 * D), lambda i: (i, 0)),
            pl.BlockSpec((BS, 2 * D), lambda i: (i, 0)),
            pl.BlockSpec((BS, 2 * D), lambda i: (i, 0)),
            pl.BlockSpec((BS, 2 * D), lambda i: (i, 0)),
            pl.BlockSpec((2 * D, 2 * D), lambda i: (0, 0)),
            pl.BlockSpec((1, 2 * D), lambda i: (0, 0)),
            pl.BlockSpec((2 * D, 2), lambda i: (0, 0)),
        ],
        out_specs=pl.BlockSpec((1, 1), lambda i: (0, 0)),
        out_shape=jax.ShapeDtypeStruct((1, 1), jnp.float32),
    )(pair, vc0, vc1, vc2, Wcat, bcat, H)


def kernel(pos_u, pos_v, neg_v, u_emb, W, b, v_emb):
    pos_u = pos_u.astype(jnp.int32)
    pos_v = pos_v.astype(jnp.int32)
    neg_v = neg_v.astype(jnp.int32)

    pu = pos_u.T.reshape(-1)
    ng = neg_v.T.reshape(-1)

    pu, ng, pos_v, u_t, v_t = lax.optimization_barrier(
        (pu, ng, pos_v, u_emb.T, v_emb.T))

    eye = jnp.eye(D, dtype=jnp.float32)
    zero = jnp.zeros((D, D), jnp.float32)
    lo = jnp.concatenate([eye, zero], axis=1)
    hi = jnp.concatenate([zero, eye], axis=1)

    v_tab = _repack(v_t, lo, hi).reshape(2 * OFF, D)
    o1, o2, o3 = _sc_gather_v(pos_v, ng, v_tab)
    u_tab = _repack(u_t, lo, hi).reshape(2 * OFF, D)
    o0 = _sc_gather_u(pu, u_tab)

    pair = o0.reshape(B, 2 * D)
    vc0 = o1.reshape(B, 2 * D)
    vc1 = o2.reshape(B, 2 * D)
    vc2 = o3.reshape(B, 2 * D)

    Wcat = jnp.concatenate([W, W], axis=0).astype(jnp.float32)
    bcat = jnp.concatenate([b, b]).reshape(1, 2 * D)
    col = jnp.arange(2 * D) >= D
    H = jnp.stack([(~col).astype(jnp.float32), col.astype(jnp.float32)],
                  axis=1)

    out = _tc_loss(pair, vc0, vc1, vc2, Wcat, bcat, H)
    return -out[0, 0]


# speedup vs baseline: 1.0022x; 1.0022x over previous
import jax
import jax.numpy as jnp
from jax import lax
from jax.experimental import pallas as pl
from jax.experimental.pallas import tpu as pltpu
from jax.experimental.pallas import tpu_sc as plsc

B = 16384
V = 100000
D = 64
PD = 64
NNEG = 5

NC = 2
NS = 16
NW = NC * NS
EPT = B // NW
CH = 512
L = 16


OB = 8192
OFF = 7 * OB


def _repack_kernel(t1_ref, t2_ref, lo_ref, hi_ref, out_ref):
    t1 = lax.dot_general(t1_ref[...], lo_ref[...], (((0,), (0,)), ((), ())),
                         preferred_element_type=jnp.float32)
    t2 = lax.dot_general(t2_ref[...], hi_ref[...], (((0,), (0,)), ((), ())),
                         preferred_element_type=jnp.float32)
    lane = lax.broadcasted_iota(jnp.int32, (OB, 2 * D), 1)
    out_ref[...] = jnp.where(lane >= D, t2, t1)


@jax.jit
def _repack(tab_t, lo, hi):
    return pl.pallas_call(
        _repack_kernel,
        grid=(OFF // OB,),
        in_specs=[pl.BlockSpec((D, OB), lambda i: (0, i)),
                  pl.BlockSpec((D, OB),
                               lambda i: (0, jnp.minimum(i + OFF // OB,
                                                         (V - 1) // OB))),
                  pl.BlockSpec((D, 2 * D), lambda i: (0, 0)),
                  pl.BlockSpec((D, 2 * D), lambda i: (0, 0))],
        out_specs=pl.BlockSpec((OB, 2 * D), lambda i: (i, 0)),
        out_shape=jax.ShapeDtypeStruct((OFF, 2 * D), jnp.float32),
    )(tab_t, tab_t, lo, hi)


def _remap(idx):
    two = idx + idx
    return jnp.where(idx >= OFF, two - (2 * OFF - 1), two)


def _sc_gather_u_kernel(pu, u_tab, pair_out, pubuf, idxbuf,
                        rows_a, rows_b, gs0, gs1, ws0, ws1):
    wid = lax.axis_index("s") * NC + lax.axis_index("c")
    e0 = wid * EPT

    pltpu.sync_copy(pu.at[pl.ds(e0, EPT)], pubuf.at[pl.ds(0, EPT)])
    pltpu.sync_copy(pu.at[pl.ds(B + e0, EPT)], pubuf.at[pl.ds(EPT, EPT)])

    lane = lax.iota(jnp.int32, L)
    for j in range(2 * EPT // L):
        p = lane + j * L
        i = p >> 1
        slot = p & 1
        idxbuf[pl.ds(j * L, L)] = _remap(
            plsc.load_gather(pubuf, [i + slot * EPT]))

    base = wid * 2 * EPT
    bufs = (rows_a, rows_b)
    gsems = (gs0, gs1)
    wsems = (ws0, ws1)
    g0 = pltpu.async_copy(u_tab.at[idxbuf.at[pl.ds(0, CH)]], rows_a, gs0)
    g1 = pltpu.async_copy(u_tab.at[idxbuf.at[pl.ds(CH, CH)]], rows_b, gs1)
    g0.wait()
    w0 = pltpu.async_copy(rows_a, pair_out.at[pl.ds(base, CH)], ws0)
    g1.wait()
    w1 = pltpu.async_copy(rows_b, pair_out.at[pl.ds(base + CH, CH)], ws1)
    w0.wait()
    w1.wait()


@jax.jit
def _sc_gather_u(pu, u_tab):
    mesh = plsc.VectorSubcoreMesh(core_axis_name="c", subcore_axis_name="s")
    f = pl.kernel(
        _sc_gather_u_kernel,
        name="sc_gather_u",
        out_type=jax.ShapeDtypeStruct((2 * B, D), jnp.float32),
        mesh=mesh,
        scratch_types=[
            pltpu.VMEM((2 * EPT,), jnp.int32),
            pltpu.VMEM((2 * EPT,), jnp.int32),
            pltpu.VMEM((CH, D), jnp.float32),
            pltpu.VMEM((CH, D), jnp.float32),
            pltpu.SemaphoreType.DMA,
            pltpu.SemaphoreType.DMA,
            pltpu.SemaphoreType.DMA,
            pltpu.SemaphoreType.DMA,
        ],
        compiler_params=pltpu.CompilerParams(use_tc_tiling_on_sc=False,
                                             needs_layout_passes=False),
    )
    return f(pu, u_tab)


def _sc_gather_v_kernel(pv, ng, v_tab, vc0_out, vc1_out, vc2_out,
                        vbuf, idxbuf, rows_a, rows_b, gs0, gs1, ws0, ws1):
    wid = lax.axis_index("s") * NC + lax.axis_index("c")
    e0 = wid * EPT

    pltpu.sync_copy(pv.at[pl.ds(e0, EPT)], vbuf.at[pl.ds(0, EPT)])
    for j in range(NNEG):
        pltpu.sync_copy(ng.at[pl.ds(j * B + e0, EPT)],
                        vbuf.at[pl.ds((1 + j) * EPT, EPT)])

    lane = lax.iota(jnp.int32, L)
    for j in range(2 * EPT // L):
        p = lane + j * L
        i = p >> 1
        slot = p & 1
        a0 = i + slot * EPT
        a1 = i + (2 + slot) * EPT
        a2 = i + (4 + slot) * EPT
        idxbuf[pl.ds(j * L, L)] = _remap(plsc.load_gather(vbuf, [a0]))
        idxbuf[pl.ds(2 * EPT + j * L, L)] = _remap(
            plsc.load_gather(vbuf, [a1]))
        idxbuf[pl.ds(4 * EPT + j * L, L)] = _remap(
            plsc.load_gather(vbuf, [a2]))

    outs = (vc0_out, vc1_out, vc2_out)
    bufs = (rows_a, rows_b)
    gsems = (gs0, gs1)
    wsems = (ws0, ws1)
    NCHV = 6 * EPT // CH
    gds, wds = [None] * NCHV, [None] * NCHV
    for c in range(NCHV):
        b = c % 2
        if c >= 2:
            wds[c - 2].wait()
        gds[c] = pltpu.async_copy(
            v_tab.at[idxbuf.at[pl.ds(c * CH, CH)]], bufs[b], gsems[b])
        if c >= 1:
            gds[c - 1].wait()
            pc = c - 1
            seg, cc = pc // 2, pc % 2
            dst = wid * 2 * EPT + cc * CH
            wds[pc] = pltpu.async_copy(
                bufs[pc % 2], outs[seg].at[pl.ds(dst, CH)], wsems[pc % 2])
    gds[NCHV - 1].wait()
    seg, cc = (NCHV - 1) // 2, (NCHV - 1) % 2
    dst = wid * 2 * EPT + cc * CH
    wds[NCHV - 1] = pltpu.async_copy(
        bufs[(NCHV - 1) % 2], outs[seg].at[pl.ds(dst, CH)],
        wsems[(NCHV - 1) % 2])
    wds[NCHV - 2].wait()
    wds[NCHV - 1].wait()


@jax.jit
def _sc_gather_v(pv, ng, v_tab):
    mesh = plsc.VectorSubcoreMesh(core_axis_name="c", subcore_axis_name="s")
    f = pl.kernel(
        _sc_gather_v_kernel,
        out_type=[jax.ShapeDtypeStruct((2 * B, D), jnp.float32)
                  for _ in range(3)],
        mesh=mesh,
        scratch_types=[
            pltpu.VMEM((6 * EPT,), jnp.int32),
            pltpu.VMEM((6 * EPT,), jnp.int32),
            pltpu.VMEM((CH, D), jnp.float32),
            pltpu.VMEM((CH, D), jnp.float32),
            pltpu.SemaphoreType.DMA,
            pltpu.SemaphoreType.DMA,
            pltpu.SemaphoreType.DMA,
            pltpu.SemaphoreType.DMA,
        ],
        compiler_params=pltpu.CompilerParams(use_tc_tiling_on_sc=False,
                                             needs_layout_passes=False),
    )
    return f(pv, ng, v_tab)


def _log_sigmoid(x):
    return jnp.minimum(x, 0.0) - jnp.log1p(jnp.exp(-jnp.abs(x)))


BS = 4096


def _tc_loss_kernel(pair_ref, vc0_ref, vc1_ref, vc2_ref, Wcat_ref, bcat_ref,
                    H_ref, out_ref):
    i = pl.program_id(0)

    relcat = lax.dot_general(pair_ref[...], Wcat_ref[...],
                             (((1,), (1,)), ((), ())),
                             preferred_element_type=jnp.float32)
    relcat += bcat_ref[...]

    pcat = jnp.concatenate([vc0_ref[...] * relcat,
                            vc1_ref[...] * relcat,
                            vc2_ref[...] * relcat], axis=0)
    st = lax.dot_general(H_ref[...], pcat, (((0,), (1,)), ((), ())),
                         preferred_element_type=jnp.float32)

    row = lax.broadcasted_iota(jnp.int32, (2, 3 * BS), 0)
    colid = lax.broadcasted_iota(jnp.int32, (2, 3 * BS), 1)
    sgn = jnp.where((row == 0) & (colid < BS), 1.0, -1.0)
    tot = jnp.sum(_log_sigmoid(st * sgn))

    @pl.when(i == 0)
    def _():
        out_ref[...] = jnp.zeros((1, 1), jnp.float32)

    out_ref[...] += jnp.broadcast_to(tot, (1, 1))


@jax.jit
def _tc_loss(pair, vc0, vc1, vc2, Wcat, bcat, H):
    return pl.pallas_call(
        _tc_loss_kernel,
        grid=(B // BS,),
        in_specs=[
            pl.BlockSpec((BS, 2 * D), lambda i: (i, 0)),
            pl.BlockSpec((BS, 2 * D), lambda i: (i, 0)),
            pl.BlockSpec((BS, 2 * D), lambda i: (i, 0)),
            pl.BlockSpec((BS, 2 * D), lambda i: (i, 0)),
            pl.BlockSpec((2 * D, 2 * D), lambda i: (0, 0)),
            pl.BlockSpec((1, 2 * D), lambda i: (0, 0)),
            pl.BlockSpec((2 * D, 2), lambda i: (0, 0)),
        ],
        out_specs=pl.BlockSpec((1, 1), lambda i: (0, 0)),
        out_shape=jax.ShapeDtypeStruct((1, 1), jnp.float32),
    )(pair, vc0, vc1, vc2, Wcat, bcat, H)


def kernel(pos_u, pos_v, neg_v, u_emb, W, b, v_emb):
    pos_u = pos_u.astype(jnp.int32)
    pos_v = pos_v.astype(jnp.int32)
    neg_v = neg_v.astype(jnp.int32)

    pu = pos_u.T.reshape(-1)
    ng = neg_v.T.reshape(-1)

    pu, ng, pos_v, u_t, v_t = lax.optimization_barrier(
        (pu, ng, pos_v, u_emb.T, v_emb.T))

    eye = jnp.eye(D, dtype=jnp.float32)
    zero = jnp.zeros((D, D), jnp.float32)
    lo = jnp.concatenate([eye, zero], axis=1)
    hi = jnp.concatenate([zero, eye], axis=1)

    v_tab = _repack(v_t, lo, hi).reshape(2 * OFF, D)
    o1, o2, o3 = _sc_gather_v(pos_v, ng, v_tab)
    u_tab = _repack(u_t, lo, hi).reshape(2 * OFF, D)
    o0 = _sc_gather_u(pu, u_tab)

    pair = o0.reshape(B, 2 * D)
    vc0 = o1.reshape(B, 2 * D)
    vc1 = o2.reshape(B, 2 * D)
    vc2 = o3.reshape(B, 2 * D)

    Wcat = jnp.concatenate([W, W], axis=0).astype(jnp.float32)
    bcat = jnp.concatenate([b, b]).reshape(1, 2 * D)
    col = jnp.arange(2 * D) >= D
    H = jnp.stack([(~col).astype(jnp.float32), col.astype(jnp.float32)],
                  axis=1)

    out = _tc_loss(pair, vc0, vc1, vc2, Wcat, bcat, H)
    return -out[0, 0]


# speedup vs baseline: 1.0047x; 1.0025x over previous
import jax
import jax.numpy as jnp
from jax import lax
from jax.experimental import pallas as pl
from jax.experimental.pallas import tpu as pltpu
from jax.experimental.pallas import tpu_sc as plsc

B = 16384
V = 100000
D = 64
PD = 64
NNEG = 5

NC = 2
NS = 16
NW = NC * NS
EPT = B // NW
CH = 512
L = 16


OB = 8192
OFF = 7 * OB


def _repack_kernel(t1_ref, t2_ref, lo_ref, hi_ref, out_ref):
    t1 = lax.dot_general(t1_ref[...], lo_ref[...], (((0,), (0,)), ((), ())),
                         preferred_element_type=jnp.float32)
    t2 = lax.dot_general(t2_ref[...], hi_ref[...], (((0,), (0,)), ((), ())),
                         preferred_element_type=jnp.float32)
    lane = lax.broadcasted_iota(jnp.int32, (OB, 2 * D), 1)
    out_ref[...] = jnp.where(lane >= D, t2, t1)


@jax.jit
def _repack(tab_t, lo, hi):
    return pl.pallas_call(
        _repack_kernel,
        grid=(OFF // OB,),
        in_specs=[pl.BlockSpec((D, OB), lambda i: (0, i)),
                  pl.BlockSpec((D, OB),
                               lambda i: (0, jnp.minimum(i + OFF // OB,
                                                         (V - 1) // OB))),
                  pl.BlockSpec((D, 2 * D), lambda i: (0, 0)),
                  pl.BlockSpec((D, 2 * D), lambda i: (0, 0))],
        out_specs=pl.BlockSpec((OB, 2 * D), lambda i: (i, 0)),
        out_shape=jax.ShapeDtypeStruct((OFF, 2 * D), jnp.float32),
    )(tab_t, tab_t, lo, hi)


def _remap(idx):
    two = idx + idx
    return jnp.where(idx >= OFF, two - (2 * OFF - 1), two)


def _sc_gather_u_kernel(pu, u_tab, pair_out, pubuf, idxbuf,
                        rows_a, rows_b, gs0, gs1, ws0, ws1):
    wid = lax.axis_index("s") * NC + lax.axis_index("c")
    e0 = wid * EPT

    pltpu.sync_copy(pu.at[pl.ds(e0, EPT)], pubuf.at[pl.ds(0, EPT)])
    pltpu.sync_copy(pu.at[pl.ds(B + e0, EPT)], pubuf.at[pl.ds(EPT, EPT)])

    lane = lax.iota(jnp.int32, L)
    for j in range(2 * EPT // L):
        p = lane + j * L
        i = p >> 1
        slot = p & 1
        idxbuf[pl.ds(j * L, L)] = _remap(
            plsc.load_gather(pubuf, [i + slot * EPT]))

    base = wid * 2 * EPT
    g0 = pltpu.async_copy(u_tab.at[idxbuf.at[pl.ds(0, CH)]], rows_a, gs0)
    g1 = pltpu.async_copy(u_tab.at[idxbuf.at[pl.ds(CH, CH)]], rows_b, gs1)
    g0.wait()
    w0 = pltpu.async_copy(rows_a, pair_out.at[pl.ds(base, CH)], ws0)
    g1.wait()
    w1 = pltpu.async_copy(rows_b, pair_out.at[pl.ds(base + CH, CH)], ws1)
    w0.wait()
    w1.wait()


@jax.jit
def _sc_gather_u(pu, u_tab):
    mesh = plsc.VectorSubcoreMesh(core_axis_name="c", subcore_axis_name="s")
    f = pl.kernel(
        _sc_gather_u_kernel,
        name="sc_gather_u",
        out_type=jax.ShapeDtypeStruct((2 * B, D), jnp.float32),
        mesh=mesh,
        scratch_types=[
            pltpu.VMEM((2 * EPT,), jnp.int32),
            pltpu.VMEM((2 * EPT,), jnp.int32),
            pltpu.VMEM((CH, D), jnp.float32),
            pltpu.VMEM((CH, D), jnp.float32),
            pltpu.SemaphoreType.DMA,
            pltpu.SemaphoreType.DMA,
            pltpu.SemaphoreType.DMA,
            pltpu.SemaphoreType.DMA,
        ],
        compiler_params=pltpu.CompilerParams(use_tc_tiling_on_sc=False,
                                             needs_layout_passes=False),
    )
    return f(pu, u_tab)


def _sc_gather_v_kernel(pv, ng, v_tab, vc0_out, vc1_out, vc2_out,
                        vbuf, idxbuf, rows_a, rows_b, gs0, gs1, ws0, ws1):
    wid = lax.axis_index("s") * NC + lax.axis_index("c")
    e0 = wid * EPT

    pltpu.sync_copy(pv.at[pl.ds(e0, EPT)], vbuf.at[pl.ds(0, EPT)])
    for j in range(NNEG):
        pltpu.sync_copy(ng.at[pl.ds(j * B + e0, EPT)],
                        vbuf.at[pl.ds((1 + j) * EPT, EPT)])

    lane = lax.iota(jnp.int32, L)
    for j in range(2 * EPT // L):
        p = lane + j * L
        i = p >> 1
        slot = p & 1
        a0 = i + slot * EPT
        a1 = i + (2 + slot) * EPT
        a2 = i + (4 + slot) * EPT
        idxbuf[pl.ds(j * L, L)] = _remap(plsc.load_gather(vbuf, [a0]))
        idxbuf[pl.ds(2 * EPT + j * L, L)] = _remap(
            plsc.load_gather(vbuf, [a1]))
        idxbuf[pl.ds(4 * EPT + j * L, L)] = _remap(
            plsc.load_gather(vbuf, [a2]))

    outs = (vc0_out, vc1_out, vc2_out)
    bufs = (rows_a, rows_b)
    gsems = (gs0, gs1)
    wsems = (ws0, ws1)
    NCHV = 6 * EPT // CH
    gds, wds = [None] * NCHV, [None] * NCHV
    for c in range(NCHV):
        b = c % 2
        if c >= 2:
            wds[c - 2].wait()
        gds[c] = pltpu.async_copy(
            v_tab.at[idxbuf.at[pl.ds(c * CH, CH)]], bufs[b], gsems[b])
        if c >= 1:
            gds[c - 1].wait()
            pc = c - 1
            seg, cc = pc // 2, pc % 2
            dst = wid * 2 * EPT + cc * CH
            wds[pc] = pltpu.async_copy(
                bufs[pc % 2], outs[seg].at[pl.ds(dst, CH)], wsems[pc % 2])
    gds[NCHV - 1].wait()
    seg, cc = (NCHV - 1) // 2, (NCHV - 1) % 2
    dst = wid * 2 * EPT + cc * CH
    wds[NCHV - 1] = pltpu.async_copy(
        bufs[(NCHV - 1) % 2], outs[seg].at[pl.ds(dst, CH)],
        wsems[(NCHV - 1) % 2])
    wds[NCHV - 2].wait()
    wds[NCHV - 1].wait()


@jax.jit
def _sc_gather_v(pv, ng, v_tab):
    mesh = plsc.VectorSubcoreMesh(core_axis_name="c", subcore_axis_name="s")
    f = pl.kernel(
        _sc_gather_v_kernel,
        out_type=[jax.ShapeDtypeStruct((2 * B, D), jnp.float32)
                  for _ in range(3)],
        mesh=mesh,
        scratch_types=[
            pltpu.VMEM((6 * EPT,), jnp.int32),
            pltpu.VMEM((6 * EPT,), jnp.int32),
            pltpu.VMEM((CH, D), jnp.float32),
            pltpu.VMEM((CH, D), jnp.float32),
            pltpu.SemaphoreType.DMA,
            pltpu.SemaphoreType.DMA,
            pltpu.SemaphoreType.DMA,
            pltpu.SemaphoreType.DMA,
        ],
        compiler_params=pltpu.CompilerParams(use_tc_tiling_on_sc=False,
                                             needs_layout_passes=False),
    )
    return f(pv, ng, v_tab)


def _log_sigmoid(x):
    return jnp.minimum(x, 0.0) - jnp.log1p(jnp.exp(-jnp.abs(x)))


BS = 4096


def _tc_loss_kernel(pair_ref, vc0_ref, vc1_ref, vc2_ref, Wcat_ref, bcat_ref,
                    H_ref, out_ref):
    i = pl.program_id(0)

    relcat = lax.dot_general(pair_ref[...], Wcat_ref[...],
                             (((1,), (1,)), ((), ())),
                             preferred_element_type=jnp.float32)
    relcat += bcat_ref[...]

    pcat = jnp.concatenate([vc0_ref[...] * relcat,
                            vc1_ref[...] * relcat,
                            vc2_ref[...] * relcat], axis=0)
    st = lax.dot_general(H_ref[...], pcat, (((0,), (1,)), ((), ())),
                         preferred_element_type=jnp.float32)

    row = lax.broadcasted_iota(jnp.int32, (2, 3 * BS), 0)
    colid = lax.broadcasted_iota(jnp.int32, (2, 3 * BS), 1)
    sgn = jnp.where((row == 0) & (colid < BS), 1.0, -1.0)
    tot = jnp.sum(_log_sigmoid(st * sgn))

    @pl.when(i == 0)
    def _():
        out_ref[...] = jnp.zeros((1, 1), jnp.float32)

    out_ref[...] += jnp.broadcast_to(tot, (1, 1))


@jax.jit
def _tc_loss(pair, vc0, vc1, vc2, Wcat, bcat, H):
    return pl.pallas_call(
        _tc_loss_kernel,
        grid=(B // BS,),
        in_specs=[
            pl.BlockSpec((BS, 2 * D), lambda i: (i, 0)),
            pl.BlockSpec((BS, 2 * D), lambda i: (i, 0)),
            pl.BlockSpec((BS, 2 * D), lambda i: (i, 0)),
            pl.BlockSpec((BS, 2 * D), lambda i: (i, 0)),
            pl.BlockSpec((2 * D, 2 * D), lambda i: (0, 0)),
            pl.BlockSpec((1, 2 * D), lambda i: (0, 0)),
            pl.BlockSpec((2 * D, 2), lambda i: (0, 0)),
        ],
        out_specs=pl.BlockSpec((1, 1), lambda i: (0, 0)),
        out_shape=jax.ShapeDtypeStruct((1, 1), jnp.float32),
    )(pair, vc0, vc1, vc2, Wcat, bcat, H)


def kernel(pos_u, pos_v, neg_v, u_emb, W, b, v_emb):
    pos_u = pos_u.astype(jnp.int32)
    pos_v = pos_v.astype(jnp.int32)
    neg_v = neg_v.astype(jnp.int32)

    pu = pos_u.T.reshape(-1)
    ng = neg_v.T.reshape(-1)

    pu, ng, pos_v, u_t, v_t = lax.optimization_barrier(
        (pu, ng, pos_v, u_emb.T, v_emb.T))

    eye = jnp.eye(D, dtype=jnp.float32)
    zero = jnp.zeros((D, D), jnp.float32)
    lo = jnp.concatenate([eye, zero], axis=1)
    hi = jnp.concatenate([zero, eye], axis=1)

    v_tab = _repack(v_t, lo, hi).reshape(2 * OFF, D)
    o1, o2, o3 = _sc_gather_v(pos_v, ng, v_tab)
    u_tab = _repack(u_t, lo, hi).reshape(2 * OFF, D)
    o0 = _sc_gather_u(pu, u_tab)

    pair = o0.reshape(B, 2 * D)
    vc0 = o1.reshape(B, 2 * D)
    vc1 = o2.reshape(B, 2 * D)
    vc2 = o3.reshape(B, 2 * D)

    Wcat = jnp.concatenate([W, W], axis=0).astype(jnp.float32)
    bcat = jnp.concatenate([b, b]).reshape(1, 2 * D)
    col = jnp.arange(2 * D) >= D
    H = jnp.stack([(~col).astype(jnp.float32), col.astype(jnp.float32)],
                  axis=1)

    out = _tc_loss(pair, vc0, vc1, vc2, Wcat, bcat, H)
    return -out[0, 0]
